# baseline (speedup 1.0000x reference)
import jax
import jax.numpy as jnp
from jax import lax
from jax.experimental import pallas as pl
from jax.experimental.pallas import tpu as pltpu

_NUM_EXPERTS = 64
_TOP_K = 8
_HIDDEN = 2048
_NUM_TOKENS = 16384
_BLK = 2048


def _router_body(x_ref, wt_ref, wout_ref, iout_ref, aux_ref,
                 cnt_ref, psum_ref):
    i = pl.program_id(0)
    nblocks = pl.num_programs(0)

    logits = lax.dot_general(
        wt_ref[...], x_ref[...],
        dimension_numbers=(((1,), (1,)), ((), ())),
        preferred_element_type=jnp.float32)
    m = jnp.max(logits, axis=0, keepdims=True)
    e = jnp.exp(logits - m)
    s = jnp.sum(e, axis=0, keepdims=True)

    eidx = lax.broadcasted_iota(jnp.int32, (_NUM_EXPERTS, _BLK), 0)

    cur = e
    sel = jnp.zeros((_NUM_EXPERTS, _BLK), jnp.float32)
    vals = []
    idxs = []
    for _ in range(_TOP_K):
        mx = jnp.max(cur, axis=0, keepdims=True)
        hit = cur == mx
        amx = jnp.min(jnp.where(hit, eidx, _NUM_EXPERTS),
                      axis=0, keepdims=True)
        pick = eidx == amx
        vals.append(mx)
        idxs.append(amx)
        sel = jnp.where(pick, 1.0, sel)
        cur = jnp.where(pick, -1.0, cur)

    w8 = jnp.concatenate(vals, axis=0)
    i8 = jnp.concatenate(idxs, axis=0)
    wout_ref[...] = w8 / jnp.sum(w8, axis=0, keepdims=True)
    iout_ref[...] = i8

    @pl.when(i == 0)
    def _init():
        cnt_ref[...] = jnp.zeros_like(cnt_ref)
        psum_ref[...] = jnp.zeros_like(psum_ref)
        aux_ref[...] = jnp.zeros((1, 1), jnp.float32)

    ones_row = jnp.ones((1, _BLK), jnp.float32)
    cnt_ref[...] += lax.dot_general(
        ones_row, sel, dimension_numbers=(((1,), (1,)), ((), ())),
        preferred_element_type=jnp.float32)
    psum_ref[...] += lax.dot_general(
        1.0 / s, e, dimension_numbers=(((1,), (1,)), ((), ())),
        preferred_element_type=jnp.float32)

    @pl.when(i == nblocks - 1)
    def _fin():
        f = cnt_ref[...] / (_NUM_TOKENS * _TOP_K)
        p = psum_ref[...] / _NUM_TOKENS
        aux_ref[...] = _NUM_EXPERTS * jnp.sum(f * p, keepdims=True).reshape(1, 1)


def kernel(hidden_states, gate_w):
    nblocks = _NUM_TOKENS // _BLK
    wt = gate_w.T
    wout_t, iout_t, aux = pl.pallas_call(
        _router_body,
        grid=(nblocks,),
        in_specs=[
            pl.BlockSpec((_BLK, _HIDDEN), lambda i: (i, 0)),
            pl.BlockSpec((_NUM_EXPERTS, _HIDDEN), lambda i: (0, 0)),
        ],
        out_specs=[
            pl.BlockSpec((_TOP_K, _BLK), lambda i: (0, i)),
            pl.BlockSpec((_TOP_K, _BLK), lambda i: (0, i)),
            pl.BlockSpec((1, 1), lambda i: (0, 0)),
        ],
        out_shape=[
            jax.ShapeDtypeStruct((_TOP_K, _NUM_TOKENS), jnp.float32),
            jax.ShapeDtypeStruct((_TOP_K, _NUM_TOKENS), jnp.int32),
            jax.ShapeDtypeStruct((1, 1), jnp.float32),
        ],
        scratch_shapes=[
            pltpu.VMEM((1, _NUM_EXPERTS), jnp.float32),
            pltpu.VMEM((1, _NUM_EXPERTS), jnp.float32),
        ],
    )(hidden_states, wt)
    return (wout_t.T, iout_t.T, aux[0, 0])


# speedup vs baseline: 1.0068x; 1.0068x over previous
import jax
import jax.numpy as jnp
from jax import lax
from jax.experimental import pallas as pl
from jax.experimental.pallas import tpu as pltpu

_NUM_EXPERTS = 64
_TOP_K = 8
_HIDDEN = 2048
_NUM_TOKENS = 16384
_BLK = 2048


def _router_body(x_ref, wt_ref, wout_ref, iout_ref, aux_ref,
                 cnt_ref, psum_ref):
    i = pl.program_id(0)
    nblocks = pl.num_programs(0)

    logits = lax.dot_general(
        wt_ref[...], x_ref[...],
        dimension_numbers=(((1,), (1,)), ((), ())),
        preferred_element_type=jnp.float32)
    m = jnp.max(logits, axis=0, keepdims=True)
    e = jnp.exp(logits - m)
    s = jnp.sum(e, axis=0, keepdims=True)

    eidx = lax.broadcasted_iota(jnp.int32, (_NUM_EXPERTS, _BLK), 0)

    cur = e
    sel = jnp.zeros((_NUM_EXPERTS, _BLK), jnp.float32)
    vals = []
    idxs = []
    for _ in range(_TOP_K):
        mx = jnp.max(cur, axis=0, keepdims=True)
        hit = cur == mx
        amx = jnp.min(jnp.where(hit, eidx, _NUM_EXPERTS),
                      axis=0, keepdims=True)
        pick = eidx == amx
        vals.append(mx)
        idxs.append(amx)
        sel = jnp.where(pick, 1.0, sel)
        cur = jnp.where(pick, -1.0, cur)

    w8 = jnp.concatenate(vals, axis=0)
    i8 = jnp.concatenate(idxs, axis=0)
    wout_ref[...] = w8 / jnp.sum(w8, axis=0, keepdims=True)
    iout_ref[...] = i8

    probs = e * (1.0 / s)

    @pl.when(i == 0)
    def _init():
        cnt_ref[...] = jnp.zeros_like(cnt_ref)
        psum_ref[...] = jnp.zeros_like(psum_ref)
        aux_ref[...] = jnp.zeros((1, 1), jnp.float32)

    cnt_ref[...] += jnp.sum(sel, axis=1, keepdims=True)
    psum_ref[...] += jnp.sum(probs, axis=1, keepdims=True)

    @pl.when(i == nblocks - 1)
    def _fin():
        f = cnt_ref[...] / (_NUM_TOKENS * _TOP_K)
        p = psum_ref[...] / _NUM_TOKENS
        aux_ref[...] = _NUM_EXPERTS * jnp.sum(f * p, keepdims=True).reshape(1, 1)


def kernel(hidden_states, gate_w):
    nblocks = _NUM_TOKENS // _BLK
    wt = gate_w.T
    wout_t, iout_t, aux = pl.pallas_call(
        _router_body,
        grid=(nblocks,),
        in_specs=[
            pl.BlockSpec((_BLK, _HIDDEN), lambda i: (i, 0)),
            pl.BlockSpec((_NUM_EXPERTS, _HIDDEN), lambda i: (0, 0)),
        ],
        out_specs=[
            pl.BlockSpec((_TOP_K, _BLK), lambda i: (0, i)),
            pl.BlockSpec((_TOP_K, _BLK), lambda i: (0, i)),
            pl.BlockSpec((1, 1), lambda i: (0, 0)),
        ],
        out_shape=[
            jax.ShapeDtypeStruct((_TOP_K, _NUM_TOKENS), jnp.float32),
            jax.ShapeDtypeStruct((_TOP_K, _NUM_TOKENS), jnp.int32),
            jax.ShapeDtypeStruct((1, 1), jnp.float32),
        ],
        scratch_shapes=[
            pltpu.VMEM((_NUM_EXPERTS, 1), jnp.float32),
            pltpu.VMEM((_NUM_EXPERTS, 1), jnp.float32),
        ],
    )(hidden_states, wt)
    return (wout_t.T, iout_t.T, aux[0, 0])
